# baseline (speedup 1.0000x reference)
import functools
import jax
import jax.numpy as jnp
from jax import lax
from jax.experimental import pallas as pl
from jax.experimental.pallas import tpu as pltpu
from jax.experimental.pallas import tpu_sc as plsc

NUM_TYPES = 119
N_NODES = 100000
OUT_COLS = NUM_TYPES + 1

CHUNK = 160
N_CHUNKS = N_NODES // CHUNK
NC, NS = 2, 16
NW = NC * NS
K_MAX = -(-N_CHUNKS // NW)
G = CHUNK // 16


def _sc_body(t_hbm, s_hbm, z_hbm, out_hbm, tb, sb, buf):
    wid = lax.axis_index("s") * NC + lax.axis_index("c")
    lane = lax.iota(jnp.int32, 16)
    ones = jnp.full((16,), 1.0, jnp.float32)
    zeros = jnp.zeros((16,), jnp.float32)
    c119 = jnp.full((16,), NUM_TYPES, jnp.int32)

    pltpu.sync_copy(z_hbm, buf)

    def step(k, carry):
        c = wid + NW * k

        @pl.when(c < N_CHUNKS)
        def _():
            base = c * CHUNK
            pltpu.sync_copy(t_hbm.at[pl.ds(base, CHUNK)], tb)
            pltpu.sync_copy(s_hbm.at[pl.ds(base * 3, CHUNK * 3)], sb)
            for g in range(G):
                rows = g * 16 + lane
                t = tb[pl.ds(g * 16, 16)]
                x = plsc.load_gather(sb, [rows * 3])
                y = plsc.load_gather(sb, [rows * 3 + 1])
                z = plsc.load_gather(sb, [rows * 3 + 2])
                s = x * x + y * y + z * z
                sn = s / jnp.maximum(s, 1e-24)
                plsc.store_scatter(buf, [rows, t], ones)
                plsc.store_scatter(buf, [rows, c119], sn)
            pltpu.sync_copy(buf, out_hbm.at[pl.ds(base, CHUNK)])
            for g in range(G):
                rows = g * 16 + lane
                t = tb[pl.ds(g * 16, 16)]
                plsc.store_scatter(buf, [rows, t], zeros)

        return carry

    lax.fori_loop(0, K_MAX, step, 0)


@functools.cache
def _sc_call():
    return pl.kernel(
        _sc_body,
        out_type=jax.ShapeDtypeStruct((N_NODES, OUT_COLS), jnp.float32),
        mesh=plsc.VectorSubcoreMesh(
            core_axis_name="c", subcore_axis_name="s", num_cores=NC, num_subcores=NS
        ),
        scratch_types=[
            pltpu.VMEM((CHUNK,), jnp.int32),
            pltpu.VMEM((CHUNK * 3,), jnp.float32),
            pltpu.VMEM((CHUNK, OUT_COLS), jnp.float32),
        ],
        compiler_params=pltpu.CompilerParams(needs_layout_passes=False, use_tc_tiling_on_sc=True),
    )


def kernel(atom_type, pos, spin):
    del pos
    t_flat = atom_type.reshape(N_NODES)
    s_flat = spin.reshape(N_NODES * 3)
    zeros_chunk = jnp.zeros((CHUNK, OUT_COLS), jnp.float32)
    node_attrs = _sc_call()(t_flat, s_flat, zeros_chunk)
    return (node_attrs, node_attrs, spin)


# speedup vs baseline: 4.6616x; 4.6616x over previous
import jax
import jax.numpy as jnp
from jax.experimental import pallas as pl

NUM_TYPES = 119
N_NODES = 100000
OUT_COLS = NUM_TYPES + 1

B = 4096
GRID = -(-N_NODES // B)
NP = GRID * B


def _body(t_ref, x_ref, y_ref, z_ref, a_ref, f_ref):
    t = t_ref[0]
    cls = jax.lax.broadcasted_iota(jnp.int32, (OUT_COLS, B), 0)
    one_hot = (cls == t).astype(jnp.float32)
    x = x_ref[0]
    y = y_ref[0]
    z = z_ref[0]
    s = x * x + y * y + z * z
    norm = jnp.sqrt(s)
    d = jnp.maximum(norm, 1e-12)
    sn = s / (d * d)
    out = jnp.where(cls == NUM_TYPES, sn, one_hot)
    a_ref[:] = out
    f_ref[:] = out


def kernel(atom_type, pos, spin):
    del pos
    pad = (0, NP - N_NODES)
    t3 = jnp.pad(atom_type.reshape(N_NODES), pad).reshape(GRID, 1, B)
    x3 = jnp.pad(spin[:, 0], pad).reshape(GRID, 1, B)
    y3 = jnp.pad(spin[:, 1], pad).reshape(GRID, 1, B)
    z3 = jnp.pad(spin[:, 2], pad).reshape(GRID, 1, B)
    in_spec = pl.BlockSpec((1, 1, B), lambda i: (i, 0, 0))
    out_spec = pl.BlockSpec((OUT_COLS, B), lambda i: (0, i))
    out_t = jax.ShapeDtypeStruct((OUT_COLS, N_NODES), jnp.float32)
    attrs_t, feats_t = pl.pallas_call(
        _body,
        grid=(GRID,),
        in_specs=[in_spec, in_spec, in_spec, in_spec],
        out_specs=[out_spec, out_spec],
        out_shape=[out_t, out_t],
    )(t3, x3, y3, z3)
    return (attrs_t.T, feats_t.T, spin)
